# baseline (speedup 1.0000x reference)
import functools

import jax
import jax.numpy as jnp
from jax import lax
from jax.experimental import pallas as pl
from jax.experimental.pallas import tpu as pltpu
from jax.experimental.pallas import tpu_sc as plsc

NUM_WINDOWS = 320
L = 16
NC, NS = 2, 16
NWORK = NC * NS
GROUP = 128


def _worker_id():
    return lax.axis_index("s") * NC + lax.axis_index("c")


def _rank_body(n, chunk, ids_hbm, destloc_hbm, hist_hbm, ids_v, destloc_v,
               counter_v):
    wid = _worker_id()
    base = wid * chunk
    pltpu.sync_copy(ids_hbm.at[pl.ds(base, chunk)], ids_v)

    iota = lax.iota(jnp.int32, L)
    zeros = jnp.zeros((L,), jnp.int32)

    def zinit(i, _):
        counter_v[pl.ds(i * L, L)] = zeros
        return 0

    lax.fori_loop(0, NUM_WINDOWS // L, zinit, 0)

    def body(j, _):
        jb = j * L
        ids = ids_v[pl.ds(jb, L)]
        rank = jnp.zeros((L,), jnp.int32)
        fwd = jnp.zeros((L,), jnp.int32)
        for k in range(1, L):
            mb = iota >= k
            prev = plsc.load_gather(ids_v, [jnp.maximum(jb + iota - k, 0)])
            rank = rank + jnp.where(mb & (prev == ids), 1, 0)
            mf = iota < L - k
            nxt = plsc.load_gather(
                ids_v, [jnp.minimum(jb + iota + k, chunk - 1)])
            fwd = fwd + jnp.where(mf & (nxt == ids), 1, 0)
        old = plsc.load_gather(counter_v, [ids])
        destloc_v[pl.ds(jb, L)] = old + rank
        plsc.store_scatter(counter_v, [ids], old + rank + 1, mask=fwd == 0)
        return 0

    lax.fori_loop(0, chunk // L, body, 0)
    pltpu.sync_copy(destloc_v, destloc_hbm.at[pl.ds(base, chunk)])
    pltpu.sync_copy(counter_v, hist_hbm.at[pl.ds(wid * NUM_WINDOWS,
                                                 NUM_WINDOWS)])


def _permute_body(n, chunk, ws, ids_hbm, destloc_hbm, hist_hbm, x_hbm,
                  idx_out_hbm, rows_out_hbm, ids_v, destloc_v, hist_v,
                  basewin_v, dest2d_v, pos_v, row_buf):
    wid = _worker_id()
    base = wid * chunk
    ngc = chunk // GROUP
    pltpu.sync_copy(ids_hbm.at[pl.ds(base, chunk)], ids_v)
    pltpu.sync_copy(destloc_hbm.at[pl.ds(base, chunk)], destloc_v)
    pltpu.sync_copy(hist_hbm, hist_v)

    iota = lax.iota(jnp.int32, L)

    def wbody(v, _):
        acc = (v * L + iota) * ws

        def cbody(c, acc):
            h = hist_v[pl.ds(c * NUM_WINDOWS + v * L, L)]
            return acc + jnp.where(c < wid, h, 0)

        acc = lax.fori_loop(0, NWORK, cbody, acc)
        basewin_v[pl.ds(v * L, L)] = acc
        return 0

    lax.fori_loop(0, NUM_WINDOWS // L, wbody, 0)

    def dbody(j, _):
        jb = j * L
        ids = ids_v[pl.ds(jb, L)]
        dl = destloc_v[pl.ds(jb, L)]
        dest = plsc.load_gather(basewin_v, [ids]) + dl
        row = j // (GROUP // L)
        col = (j % (GROUP // L)) * L + iota
        rowv = jnp.full((L,), 0, jnp.int32) + row
        plsc.store_scatter(dest2d_v, [rowv, col], dest)
        plsc.store_scatter(dest2d_v, [rowv + ngc, col], dest + n)
        pos_v[pl.ds(jb, L)] = base + jb + iota
        return 0

    lax.fori_loop(0, chunk // L, dbody, 0)

    def ibody(g, _):
        pltpu.sync_copy(pos_v.at[pl.ds(g * GROUP, GROUP)],
                        idx_out_hbm.at[dest2d_v.at[g]])
        return 0

    lax.fori_loop(0, ngc, ibody, 0)

    def rbody(t, _):
        b = t // ngc
        g = t % ngc
        src = b * n + base + g * GROUP
        pltpu.sync_copy(x_hbm.at[pl.ds(src, GROUP)], row_buf)
        pltpu.sync_copy(row_buf, rows_out_hbm.at[dest2d_v.at[t]])
        return 0

    lax.fori_loop(0, 2 * ngc, rbody, 0)


def kernel(x, window_ids):
    b, n, c = x.shape
    ws = n // NUM_WINDOWS
    chunk = n // NWORK
    x_flat = x.reshape(b * n, c)
    mesh = plsc.VectorSubcoreMesh(core_axis_name="c", subcore_axis_name="s")

    params = pltpu.CompilerParams(needs_layout_passes=False,
                                  use_tc_tiling_on_sc=False)
    rank_call = pl.kernel(
        functools.partial(_rank_body, n, chunk),
        out_type=(
            jax.ShapeDtypeStruct((n,), jnp.int32),
            jax.ShapeDtypeStruct((NWORK * NUM_WINDOWS,), jnp.int32),
        ),
        mesh=mesh,
        compiler_params=params,
        scratch_types=[
            pltpu.VMEM((chunk,), jnp.int32),
            pltpu.VMEM((chunk,), jnp.int32),
            pltpu.VMEM((NUM_WINDOWS,), jnp.int32),
        ],
    )
    destloc, hist = rank_call(window_ids)

    ngc = chunk // GROUP
    permute_call = pl.kernel(
        functools.partial(_permute_body, n, chunk, ws),
        out_type=(
            jax.ShapeDtypeStruct((n,), jnp.int32),
            jax.ShapeDtypeStruct((b * n, c), jnp.float32),
        ),
        mesh=mesh,
        compiler_params=params,
        scratch_types=[
            pltpu.VMEM((chunk,), jnp.int32),
            pltpu.VMEM((chunk,), jnp.int32),
            pltpu.VMEM((NWORK * NUM_WINDOWS,), jnp.int32),
            pltpu.VMEM((NUM_WINDOWS,), jnp.int32),
            pltpu.VMEM((2 * ngc, GROUP), jnp.int32),
            pltpu.VMEM((chunk,), jnp.int32),
            pltpu.VMEM((GROUP, c), jnp.float32),
        ],
    )
    indices, out_flat = permute_call(window_ids, destloc, hist, x_flat)

    windows = out_flat.reshape(b * NUM_WINDOWS, ws, c)
    return (windows, jnp.asarray(NUM_WINDOWS, jnp.int32), indices)


# speedup vs baseline: 1.0230x; 1.0230x over previous
import functools

import jax
import jax.numpy as jnp
from jax import lax
from jax.experimental import pallas as pl
from jax.experimental.pallas import tpu as pltpu
from jax.experimental.pallas import tpu_sc as plsc

NUM_WINDOWS = 320
L = 16
NC, NS = 2, 16
NWORK = NC * NS
GROUP = 128


def _worker_id():
    return lax.axis_index("s") * NC + lax.axis_index("c")


def _rank_body(n, chunk, ids_hbm, destloc_hbm, hist_hbm, ids_v, destloc_v,
               counter_v):
    wid = _worker_id()
    base = wid * chunk
    pltpu.sync_copy(ids_hbm.at[pl.ds(base, chunk)], ids_v)

    iota = lax.iota(jnp.int32, L)
    zeros = jnp.zeros((L,), jnp.int32)

    def zinit(i, _):
        counter_v[pl.ds(i * L, L)] = zeros
        return 0

    lax.fori_loop(0, NUM_WINDOWS // L, zinit, 0)

    def body(j, _):
        jb = j * L
        ids = ids_v[pl.ds(jb, L)]
        rank = jnp.zeros((L,), jnp.int32)
        fwd = jnp.zeros((L,), jnp.int32)
        for k in range(1, L):
            mb = iota >= k
            prev = plsc.load_gather(ids_v, [jnp.maximum(jb + iota - k, 0)])
            rank = rank + jnp.where(mb & (prev == ids), 1, 0)
            mf = iota < L - k
            nxt = plsc.load_gather(
                ids_v, [jnp.minimum(jb + iota + k, chunk - 1)])
            fwd = fwd + jnp.where(mf & (nxt == ids), 1, 0)
        old = plsc.load_gather(counter_v, [ids])
        destloc_v[pl.ds(jb, L)] = old + rank
        plsc.store_scatter(counter_v, [ids], old + rank + 1, mask=fwd == 0)
        return 0

    lax.fori_loop(0, chunk // L, body, 0)
    pltpu.sync_copy(destloc_v, destloc_hbm.at[pl.ds(base, chunk)])
    pltpu.sync_copy(counter_v, hist_hbm.at[pl.ds(wid * NUM_WINDOWS,
                                                 NUM_WINDOWS)])


def _permute_body(n, chunk, ws, ids_hbm, destloc_hbm, hist_hbm, x_hbm,
                  idx_out_hbm, rows_out_hbm, ids_v, destloc_v, hist_v,
                  basewin_v, dest2d_v, pos_v, row_buf0, row_buf1, lsem0,
                  lsem1, ssem0, ssem1):
    wid = _worker_id()
    base = wid * chunk
    ngc = chunk // GROUP
    pltpu.sync_copy(ids_hbm.at[pl.ds(base, chunk)], ids_v)
    pltpu.sync_copy(destloc_hbm.at[pl.ds(base, chunk)], destloc_v)
    pltpu.sync_copy(hist_hbm, hist_v)

    iota = lax.iota(jnp.int32, L)

    def wbody(v, _):
        acc = (v * L + iota) * ws

        def cbody(c, acc):
            h = hist_v[pl.ds(c * NUM_WINDOWS + v * L, L)]
            return acc + jnp.where(c < wid, h, 0)

        acc = lax.fori_loop(0, NWORK, cbody, acc)
        basewin_v[pl.ds(v * L, L)] = acc
        return 0

    lax.fori_loop(0, NUM_WINDOWS // L, wbody, 0)

    def dbody(j, _):
        jb = j * L
        ids = ids_v[pl.ds(jb, L)]
        dl = destloc_v[pl.ds(jb, L)]
        dest = plsc.load_gather(basewin_v, [ids]) + dl
        row = j // (GROUP // L)
        col = (j % (GROUP // L)) * L + iota
        rowv = jnp.full((L,), 0, jnp.int32) + row
        plsc.store_scatter(dest2d_v, [rowv, col], dest)
        plsc.store_scatter(dest2d_v, [rowv + ngc, col], dest + n)
        pos_v[pl.ds(jb, L)] = base + jb + iota
        return 0

    lax.fori_loop(0, chunk // L, dbody, 0)

    def ibody(g, _):
        pltpu.sync_copy(pos_v.at[pl.ds(g * GROUP, GROUP)],
                        idx_out_hbm.at[dest2d_v.at[g]])
        return 0

    lax.fori_loop(0, ngc, ibody, 0)

    bufs = (row_buf0, row_buf1)
    lsems = (lsem0, lsem1)
    ssems = (ssem0, ssem1)
    store_descs = [None, None]
    for t in range(2 * ngc):
        buf = t % 2
        b, g = divmod(t, ngc)
        if store_descs[buf] is not None:
            store_descs[buf].wait()
        pltpu.async_copy(
            x_hbm.at[b, pl.ds(base + g * GROUP, GROUP)], bufs[buf],
            lsems[buf]).wait()
        store_descs[buf] = pltpu.async_copy(
            bufs[buf], rows_out_hbm.at[dest2d_v.at[t]], ssems[buf])
    store_descs[0].wait()
    store_descs[1].wait()


def kernel(x, window_ids):
    b, n, c = x.shape
    ws = n // NUM_WINDOWS
    chunk = n // NWORK
    mesh = plsc.VectorSubcoreMesh(core_axis_name="c", subcore_axis_name="s")

    params = pltpu.CompilerParams(needs_layout_passes=False,
                                  use_tc_tiling_on_sc=False)
    rank_call = pl.kernel(
        functools.partial(_rank_body, n, chunk),
        out_type=(
            jax.ShapeDtypeStruct((n,), jnp.int32),
            jax.ShapeDtypeStruct((NWORK * NUM_WINDOWS,), jnp.int32),
        ),
        mesh=mesh,
        compiler_params=params,
        scratch_types=[
            pltpu.VMEM((chunk,), jnp.int32),
            pltpu.VMEM((chunk,), jnp.int32),
            pltpu.VMEM((NUM_WINDOWS,), jnp.int32),
        ],
    )
    destloc, hist = rank_call(window_ids)

    ngc = chunk // GROUP
    permute_call = pl.kernel(
        functools.partial(_permute_body, n, chunk, ws),
        out_type=(
            jax.ShapeDtypeStruct((n,), jnp.int32),
            jax.ShapeDtypeStruct((b * n, c), jnp.float32),
        ),
        mesh=mesh,
        compiler_params=params,
        scratch_types=[
            pltpu.VMEM((chunk,), jnp.int32),
            pltpu.VMEM((chunk,), jnp.int32),
            pltpu.VMEM((NWORK * NUM_WINDOWS,), jnp.int32),
            pltpu.VMEM((NUM_WINDOWS,), jnp.int32),
            pltpu.VMEM((2 * ngc, GROUP), jnp.int32),
            pltpu.VMEM((chunk,), jnp.int32),
            pltpu.VMEM((GROUP, c), jnp.float32),
            pltpu.VMEM((GROUP, c), jnp.float32),
            pltpu.SemaphoreType.DMA,
            pltpu.SemaphoreType.DMA,
            pltpu.SemaphoreType.DMA,
            pltpu.SemaphoreType.DMA,
        ],
    )
    indices, out_flat = permute_call(window_ids, destloc, hist, x)

    windows = out_flat.reshape(b * NUM_WINDOWS, ws, c)
    return (windows, jnp.asarray(NUM_WINDOWS, jnp.int32), indices)
